# baseline (speedup 1.0000x reference)
import functools

import jax
import jax.numpy as jnp
from jax import lax
from jax.experimental import pallas as pl
from jax.experimental.pallas import tpu as pltpu
from jax.experimental.pallas import tpu_sc as plsc

B2 = 32768
D = 64
NW = 32
PER_W = B2 // NW
CHUNK = 128
IDX_W = 128
LMBDA = 0.01


def _sc_gather_score(h2, offh, t2, offt, r2, offr, ent2, rel2):
    mesh = plsc.VectorSubcoreMesh(core_axis_name="c", subcore_axis_name="s")
    rows_w = PER_W // IDX_W
    rows_c = CHUNK // IDX_W

    @functools.partial(
        pl.kernel,
        mesh=mesh,
        compiler_params=pltpu.CompilerParams(
            needs_layout_passes=False, use_tc_tiling_on_sc=True),
        out_type=[
            jax.ShapeDtypeStruct((B2,), jnp.float32),
            jax.ShapeDtypeStruct((NW, 16), jnp.float32),
        ],
        scratch_types=[
            pltpu.VMEM((rows_w, IDX_W), jnp.int32),
            pltpu.VMEM((rows_w, IDX_W), jnp.int32),
            pltpu.VMEM((rows_w, IDX_W), jnp.int32),
            pltpu.VMEM((rows_w, IDX_W), jnp.int32),
            pltpu.VMEM((rows_w, IDX_W), jnp.int32),
            pltpu.VMEM((rows_w, IDX_W), jnp.int32),
            pltpu.VMEM((CHUNK, 2 * D), jnp.int32),
            pltpu.VMEM((CHUNK, 2 * D), jnp.int32),
            pltpu.VMEM((CHUNK, 2 * D), jnp.int32),
            pltpu.VMEM((CHUNK, 2 * D), jnp.int32),
            pltpu.VMEM((CHUNK, 2 * D), jnp.int32),
            pltpu.VMEM((CHUNK, 2 * D), jnp.int32),
            pltpu.VMEM((PER_W,), jnp.float32),
            pltpu.VMEM((16,), jnp.float32),
            pltpu.SemaphoreType.DMA,
            pltpu.SemaphoreType.DMA,
        ],
    )
    def sc_kernel(h2_hbm, offh_hbm, t2_hbm, offt_hbm, r2_hbm, offr_hbm,
                  ent_hbm, rel_hbm, dots_hbm, sq_hbm,
                  h2_v, offh_v, t2_v, offt_v, r2_v, offr_v,
                  h_rows0, t_rows0, r_rows0, h_rows1, t_rows1, r_rows1,
                  dots_v, sq_v, sem0, sem1):
        wid = lax.axis_index("s") * 2 + lax.axis_index("c")
        lane = lax.broadcasted_iota(jnp.int32, (16,), 0)
        base_row = wid * rows_w

        pltpu.sync_copy(h2_hbm.at[pl.ds(base_row, rows_w)], h2_v)
        pltpu.sync_copy(offh_hbm.at[pl.ds(base_row, rows_w)], offh_v)
        pltpu.sync_copy(t2_hbm.at[pl.ds(base_row, rows_w)], t2_v)
        pltpu.sync_copy(offt_hbm.at[pl.ds(base_row, rows_w)], offt_v)
        pltpu.sync_copy(r2_hbm.at[pl.ds(base_row, rows_w)], r2_v)
        pltpu.sync_copy(offr_hbm.at[pl.ds(base_row, rows_w)], offr_v)

        bufs = [(h_rows0, t_rows0, r_rows0), (h_rows1, t_rows1, r_rows1)]
        sems = [sem0, sem1]
        n_chunks = PER_W // CHUNK

        def issue(chunk):
            hb, tb, rb = bufs[chunk % 2]
            sem = sems[chunk % 2]
            return [
                pltpu.async_copy(ent_hbm.at[h2_v.at[chunk]], hb, sem),
                pltpu.async_copy(ent_hbm.at[t2_v.at[chunk]], tb, sem),
                pltpu.async_copy(rel_hbm.at[r2_v.at[chunk]], rb, sem),
            ]

        sq_acc = jnp.zeros((16,), jnp.float32)
        pending = {0: issue(0)}
        for chunk in range(n_chunks):
            for dsc in pending.pop(chunk):
                dsc.wait()
            if chunk + 1 < n_chunks:
                pending[chunk + 1] = issue(chunk + 1)
            h_rows, t_rows, r_rows = bufs[chunk % 2]

            def group_body(g, sq_acc, chunk=chunk,
                           h_rows=h_rows, t_rows=t_rows, r_rows=r_rows):
                gcol = g * 16
                ohv = offh_v[chunk, pl.ds(gcol, 16)]
                otv = offt_v[chunk, pl.ds(gcol, 16)]
                orv = offr_v[chunk, pl.ds(gcol, 16)]
                svec = jnp.zeros((16,), jnp.float32)
                for j in range(16):
                    row = g * 16 + j
                    oh, ot, orr = ohv[j], otv[j], orv[j]
                    acc = None
                    for c in range(2):
                        hw = h_rows[row, pl.ds(oh + c * 16, 16)]
                        tw = t_rows[row, pl.ds(ot + c * 16, 16)]
                        rw = r_rows[row, pl.ds(orr + c * 16, 16)]
                        h0, h1 = plsc.unpack(
                            plsc.bitcast(hw, jnp.bfloat16),
                            format=plsc.PackFormat.INTERLEAVED)
                        t0, t1 = plsc.unpack(
                            plsc.bitcast(tw, jnp.bfloat16),
                            format=plsc.PackFormat.INTERLEAVED)
                        r0, r1 = plsc.unpack(
                            plsc.bitcast(rw, jnp.bfloat16),
                            format=plsc.PackFormat.INTERLEAVED)
                        p = h0 * t0 * r0 + h1 * t1 * r1
                        acc = p if acc is None else acc + p
                        sq_acc = sq_acc + (h0 * h0 + h1 * h1 + t0 * t0
                                           + t1 * t1 + r0 * r0 + r1 * r1)
                    svec = jnp.where(lane == j, jnp.sum(acc), svec)
                dots_v[pl.ds(chunk * CHUNK + g * 16, 16)] = svec
                return sq_acc

            sq_acc = lax.fori_loop(0, CHUNK // 16, group_body, sq_acc)
        pltpu.sync_copy(dots_v, dots_hbm.at[pl.ds(wid * PER_W, PER_W)])
        sq_v[...] = sq_acc
        pltpu.sync_copy(sq_v, sq_hbm.at[wid])

    return sc_kernel(h2, offh, t2, offt, r2, offr, ent2, rel2)


_EQ = 16000
_RQ = 512


def _tc_transpose_quads(tab_t, n_rows, q):
    grid = (n_rows + 4 * q - 1) // (4 * q)

    def body(b0, b1, b2, b3, out_ref):
        words = []
        for ref in (b0, b1, b2, b3):
            u = lax.bitcast_convert_type(ref[...], jnp.uint32)
            r = (u + 0x7FFF + ((u >> 16) & 1)) >> 16
            words.append((r[32:64, :] << 16) | r[0:32, :])
        stacked = jnp.concatenate(words, axis=0)
        out_ref[...] = jnp.transpose(
            lax.bitcast_convert_type(stacked, jnp.int32))

    def qmap(k):
        def f(i):
            j = 4 * i + k
            return (0, jnp.where(j * q >= n_rows, 0, j))
        return f

    return pl.pallas_call(
        body,
        grid=(grid,),
        in_specs=[pl.BlockSpec((64, q), qmap(k)) for k in range(4)],
        out_specs=pl.BlockSpec((q, 128), lambda i: (i, 0)),
        out_shape=jax.ShapeDtypeStruct((grid * q, 128), jnp.int32),
    )(tab_t, tab_t, tab_t, tab_t)


def _finalize(dots, sq):
    rows = B2 // 128

    def body(dots_ref, sq_ref, out_ref):
        s = dots_ref[...]
        rowid = lax.broadcasted_iota(jnp.int32, (rows, 128), 0)
        x = jnp.where(rowid < rows // 2, -s, s)
        sp = jnp.maximum(x, 0.0) + jnp.log1p(jnp.exp(-jnp.abs(x)))
        mean_sp = jnp.sum(sp) / float(B2)
        regul = jnp.sum(sq_ref[...]) / float(B2 * D)
        out_ref[...] = jnp.reshape(mean_sp + LMBDA * regul, (1, 1))

    return pl.pallas_call(
        body,
        out_shape=jax.ShapeDtypeStruct((1, 1), jnp.float32),
    )(dots.reshape(rows, 128), sq)


def kernel(pos_h, pos_r, pos_t, neg_h, neg_r, neg_t, entity_emb, relation_emb):
    h_idx = jnp.concatenate([pos_h, neg_h])
    t_idx = jnp.concatenate([pos_t, neg_t])
    r_idx = jnp.concatenate([pos_r[:, 0], neg_r[:, 0]])

    def split(idx, q):
        row = (idx // (4 * q)) * q + (idx % q)
        off = ((idx // q) % 4) * 32
        return (jnp.reshape(row, (B2 // IDX_W, IDX_W)),
                jnp.reshape(off, (B2 // IDX_W, IDX_W)))

    h2, offh = split(h_idx, _EQ)
    t2, offt = split(t_idx, _EQ)
    r2, offr = split(r_idx, _RQ)
    ent2 = _tc_transpose_quads(entity_emb.T, 1000000, _EQ)
    rel2 = _tc_transpose_quads(relation_emb.T, 1000, _RQ)
    dots, sq = _sc_gather_score(h2, offh, t2, offt, r2, offr, ent2, rel2)
    return _finalize(dots, sq)[0, 0]


# speedup vs baseline: 1.0242x; 1.0242x over previous
import functools

import jax
import jax.numpy as jnp
from jax import lax
from jax.experimental import pallas as pl
from jax.experimental.pallas import tpu as pltpu
from jax.experimental.pallas import tpu_sc as plsc

B2 = 32768
D = 64
NW = 32
PER_W = B2 // NW
CHUNK = 128
IDX_W = 128
LMBDA = 0.01


def _sc_gather_score(h2, offh, t2, offt, r2, offr, ent2, rel2):
    mesh = plsc.VectorSubcoreMesh(core_axis_name="c", subcore_axis_name="s")
    rows_w = PER_W // IDX_W
    rows_c = CHUNK // IDX_W

    @functools.partial(
        pl.kernel,
        mesh=mesh,
        compiler_params=pltpu.CompilerParams(
            needs_layout_passes=False, use_tc_tiling_on_sc=True),
        out_type=[
            jax.ShapeDtypeStruct((B2,), jnp.float32),
            jax.ShapeDtypeStruct((NW, 16), jnp.float32),
        ],
        scratch_types=[
            pltpu.VMEM((rows_w, IDX_W), jnp.int32),
            pltpu.VMEM((rows_w, IDX_W), jnp.int32),
            pltpu.VMEM((rows_w, IDX_W), jnp.int32),
            pltpu.VMEM((rows_w, IDX_W), jnp.int32),
            pltpu.VMEM((rows_w, IDX_W), jnp.int32),
            pltpu.VMEM((rows_w, IDX_W), jnp.int32),
            pltpu.VMEM((CHUNK, 2 * D), jnp.int32),
            pltpu.VMEM((CHUNK, 2 * D), jnp.int32),
            pltpu.VMEM((CHUNK, 2 * D), jnp.int32),
            pltpu.VMEM((CHUNK, 2 * D), jnp.int32),
            pltpu.VMEM((_RQ, 2 * D), jnp.int32),
            pltpu.VMEM((PER_W,), jnp.float32),
            pltpu.VMEM((16,), jnp.float32),
            pltpu.SemaphoreType.DMA,
            pltpu.SemaphoreType.DMA,
        ],
    )
    def sc_kernel(h2_hbm, offh_hbm, t2_hbm, offt_hbm, r2_hbm, offr_hbm,
                  ent_hbm, rel_hbm, dots_hbm, sq_hbm,
                  h2_v, offh_v, t2_v, offt_v, r2_v, offr_v,
                  h_rows0, t_rows0, h_rows1, t_rows1, rel_cache,
                  dots_v, sq_v, sem0, sem1):
        wid = lax.axis_index("s") * 2 + lax.axis_index("c")
        lane = lax.broadcasted_iota(jnp.int32, (16,), 0)
        base_row = wid * rows_w

        pltpu.sync_copy(h2_hbm.at[pl.ds(base_row, rows_w)], h2_v)
        pltpu.sync_copy(offh_hbm.at[pl.ds(base_row, rows_w)], offh_v)
        pltpu.sync_copy(t2_hbm.at[pl.ds(base_row, rows_w)], t2_v)
        pltpu.sync_copy(offt_hbm.at[pl.ds(base_row, rows_w)], offt_v)
        pltpu.sync_copy(r2_hbm.at[pl.ds(base_row, rows_w)], r2_v)
        pltpu.sync_copy(offr_hbm.at[pl.ds(base_row, rows_w)], offr_v)
        pltpu.sync_copy(rel_hbm, rel_cache)

        bufs = [(h_rows0, t_rows0), (h_rows1, t_rows1)]
        sems = [sem0, sem1]
        n_chunks = PER_W // CHUNK

        def issue(chunk):
            hb, tb = bufs[chunk % 2]
            sem = sems[chunk % 2]
            return [
                pltpu.async_copy(ent_hbm.at[h2_v.at[chunk]], hb, sem),
                pltpu.async_copy(ent_hbm.at[t2_v.at[chunk]], tb, sem),
            ]

        sq_acc = jnp.zeros((16,), jnp.float32)
        pending = {0: issue(0)}
        for chunk in range(n_chunks):
            for dsc in pending.pop(chunk):
                dsc.wait()
            if chunk + 1 < n_chunks:
                pending[chunk + 1] = issue(chunk + 1)
            h_rows, t_rows = bufs[chunk % 2]

            def group_body(g, sq_acc, chunk=chunk,
                           h_rows=h_rows, t_rows=t_rows):
                gcol = g * 16
                ohv = offh_v[chunk, pl.ds(gcol, 16)]
                otv = offt_v[chunk, pl.ds(gcol, 16)]
                orv = offr_v[chunk, pl.ds(gcol, 16)]
                rrv = r2_v[chunk, pl.ds(gcol, 16)]
                svec = jnp.zeros((16,), jnp.float32)
                for j in range(16):
                    row = g * 16 + j
                    oh, ot, orr, rrow = ohv[j], otv[j], orv[j], rrv[j]
                    acc = None
                    for c in range(2):
                        hw = h_rows[row, pl.ds(oh + c * 16, 16)]
                        tw = t_rows[row, pl.ds(ot + c * 16, 16)]
                        rw = rel_cache[rrow, pl.ds(orr + c * 16, 16)]
                        h0, h1 = plsc.unpack(
                            plsc.bitcast(hw, jnp.bfloat16),
                            format=plsc.PackFormat.INTERLEAVED)
                        t0, t1 = plsc.unpack(
                            plsc.bitcast(tw, jnp.bfloat16),
                            format=plsc.PackFormat.INTERLEAVED)
                        r0, r1 = plsc.unpack(
                            plsc.bitcast(rw, jnp.bfloat16),
                            format=plsc.PackFormat.INTERLEAVED)
                        p = h0 * t0 * r0 + h1 * t1 * r1
                        acc = p if acc is None else acc + p
                        sq_acc = sq_acc + (h0 * h0 + h1 * h1 + t0 * t0
                                           + t1 * t1 + r0 * r0 + r1 * r1)
                    svec = jnp.where(lane == j, jnp.sum(acc), svec)
                dots_v[pl.ds(chunk * CHUNK + g * 16, 16)] = svec
                return sq_acc

            sq_acc = lax.fori_loop(0, CHUNK // 16, group_body, sq_acc)
        pltpu.sync_copy(dots_v, dots_hbm.at[pl.ds(wid * PER_W, PER_W)])
        sq_v[...] = sq_acc
        pltpu.sync_copy(sq_v, sq_hbm.at[wid])

    return sc_kernel(h2, offh, t2, offt, r2, offr, ent2, rel2)


_EQ = 12800
_RQ = 256


def _tc_transpose_quads(tab_t, n_rows, q):
    grid = (n_rows + 4 * q - 1) // (4 * q)

    def body(b0, b1, b2, b3, out_ref):
        words = []
        for ref in (b0, b1, b2, b3):
            u = lax.bitcast_convert_type(ref[...], jnp.uint32)
            r = (u + 0x7FFF + ((u >> 16) & 1)) >> 16
            words.append((r[32:64, :] << 16) | r[0:32, :])
        stacked = jnp.concatenate(words, axis=0)
        out_ref[...] = jnp.transpose(
            lax.bitcast_convert_type(stacked, jnp.int32))

    def qmap(k):
        def f(i):
            j = 4 * i + k
            return (0, jnp.where(j * q >= n_rows, 0, j))
        return f

    return pl.pallas_call(
        body,
        grid=(grid,),
        in_specs=[pl.BlockSpec((64, q), qmap(k)) for k in range(4)],
        out_specs=pl.BlockSpec((q, 128), lambda i: (i, 0)),
        out_shape=jax.ShapeDtypeStruct((grid * q, 128), jnp.int32),
    )(tab_t, tab_t, tab_t, tab_t)


def _finalize(dots, sq):
    rows = B2 // 128

    def body(dots_ref, sq_ref, out_ref):
        s = dots_ref[...]
        rowid = lax.broadcasted_iota(jnp.int32, (rows, 128), 0)
        x = jnp.where(rowid < rows // 2, -s, s)
        sp = jnp.maximum(x, 0.0) + jnp.log1p(jnp.exp(-jnp.abs(x)))
        mean_sp = jnp.sum(sp) / float(B2)
        regul = jnp.sum(sq_ref[...]) / float(B2 * D)
        out_ref[...] = jnp.reshape(mean_sp + LMBDA * regul, (1, 1))

    return pl.pallas_call(
        body,
        out_shape=jax.ShapeDtypeStruct((1, 1), jnp.float32),
    )(dots.reshape(rows, 128), sq)


def kernel(pos_h, pos_r, pos_t, neg_h, neg_r, neg_t, entity_emb, relation_emb):
    h_idx = jnp.concatenate([pos_h, neg_h])
    t_idx = jnp.concatenate([pos_t, neg_t])
    r_idx = jnp.concatenate([pos_r[:, 0], neg_r[:, 0]])

    def split(idx, q):
        row = (idx // (4 * q)) * q + (idx % q)
        off = ((idx // q) % 4) * 32
        return (jnp.reshape(row, (B2 // IDX_W, IDX_W)),
                jnp.reshape(off, (B2 // IDX_W, IDX_W)))

    h2, offh = split(h_idx, _EQ)
    t2, offt = split(t_idx, _EQ)
    r2, offr = split(r_idx, _RQ)
    ent2 = _tc_transpose_quads(entity_emb.T, 1000000, _EQ)
    rel2 = _tc_transpose_quads(relation_emb.T, 1000, _RQ)
    dots, sq = _sc_gather_score(h2, offh, t2, offt, r2, offr, ent2, rel2)
    return _finalize(dots, sq)[0, 0]


# speedup vs baseline: 1.0384x; 1.0139x over previous
import functools

import jax
import jax.numpy as jnp
from jax import lax
from jax.experimental import pallas as pl
from jax.experimental.pallas import tpu as pltpu
from jax.experimental.pallas import tpu_sc as plsc

B2 = 32768
D = 64
NW = 32
PER_W = B2 // NW
CHUNK = 128
IDX_W = 128
LMBDA = 0.01


def _sc_gather_score(h2, offh, t2, offt, r2, offr, ent2, rel2):
    mesh = plsc.VectorSubcoreMesh(core_axis_name="c", subcore_axis_name="s")
    rows_w = PER_W // IDX_W
    rows_c = CHUNK // IDX_W

    @functools.partial(
        pl.kernel,
        mesh=mesh,
        compiler_params=pltpu.CompilerParams(
            needs_layout_passes=False, use_tc_tiling_on_sc=False),
        out_type=[
            jax.ShapeDtypeStruct((B2,), jnp.float32),
            jax.ShapeDtypeStruct((NW, 16), jnp.float32),
        ],
        scratch_types=[
            pltpu.VMEM((rows_w, IDX_W), jnp.int32),
            pltpu.VMEM((rows_w, IDX_W), jnp.int32),
            pltpu.VMEM((rows_w, IDX_W), jnp.int32),
            pltpu.VMEM((rows_w, IDX_W), jnp.int32),
            pltpu.VMEM((rows_w, IDX_W), jnp.int32),
            pltpu.VMEM((rows_w, IDX_W), jnp.int32),
            pltpu.VMEM((CHUNK, D), jnp.int32),
            pltpu.VMEM((CHUNK, D), jnp.int32),
            pltpu.VMEM((CHUNK, D), jnp.int32),
            pltpu.VMEM((CHUNK, D), jnp.int32),
            pltpu.VMEM((_RQ, 2 * D), jnp.int32),
            pltpu.VMEM((PER_W,), jnp.float32),
            pltpu.VMEM((16,), jnp.float32),
            pltpu.SemaphoreType.DMA,
            pltpu.SemaphoreType.DMA,
        ],
    )
    def sc_kernel(h2_hbm, offh_hbm, t2_hbm, offt_hbm, r2_hbm, offr_hbm,
                  ent_hbm, rel_hbm, dots_hbm, sq_hbm,
                  h2_v, offh_v, t2_v, offt_v, r2_v, offr_v,
                  h_rows0, t_rows0, h_rows1, t_rows1, rel_cache,
                  dots_v, sq_v, sem0, sem1):
        wid = lax.axis_index("s") * 2 + lax.axis_index("c")
        lane = lax.broadcasted_iota(jnp.int32, (16,), 0)
        base_row = wid * rows_w

        pltpu.sync_copy(h2_hbm.at[pl.ds(base_row, rows_w)], h2_v)
        pltpu.sync_copy(offh_hbm.at[pl.ds(base_row, rows_w)], offh_v)
        pltpu.sync_copy(t2_hbm.at[pl.ds(base_row, rows_w)], t2_v)
        pltpu.sync_copy(offt_hbm.at[pl.ds(base_row, rows_w)], offt_v)
        pltpu.sync_copy(r2_hbm.at[pl.ds(base_row, rows_w)], r2_v)
        pltpu.sync_copy(offr_hbm.at[pl.ds(base_row, rows_w)], offr_v)
        pltpu.sync_copy(rel_hbm, rel_cache)

        bufs = [(h_rows0, t_rows0), (h_rows1, t_rows1)]
        sems = [sem0, sem1]
        n_chunks = PER_W // CHUNK

        def issue(chunk):
            hb, tb = bufs[chunk % 2]
            sem = sems[chunk % 2]
            return [
                pltpu.async_copy(ent_hbm.at[h2_v.at[chunk]], hb, sem),
                pltpu.async_copy(ent_hbm.at[t2_v.at[chunk]], tb, sem),
            ]

        sq_acc = jnp.zeros((16,), jnp.float32)
        pending = {0: issue(0)}
        for chunk in range(n_chunks):
            for dsc in pending.pop(chunk):
                dsc.wait()
            if chunk + 1 < n_chunks:
                pending[chunk + 1] = issue(chunk + 1)
            h_rows, t_rows = bufs[chunk % 2]

            def group_body(g, sq_acc, chunk=chunk,
                           h_rows=h_rows, t_rows=t_rows):
                gcol = g * 16
                ohv = offh_v[chunk, pl.ds(gcol, 16)]
                otv = offt_v[chunk, pl.ds(gcol, 16)]
                orv = offr_v[chunk, pl.ds(gcol, 16)]
                rrv = r2_v[chunk, pl.ds(gcol, 16)]
                svec = jnp.zeros((16,), jnp.float32)
                for j in range(16):
                    row = g * 16 + j
                    oh, ot, orr, rrow = ohv[j], otv[j], orv[j], rrv[j]
                    acc = None
                    for c in range(2):
                        hw = h_rows[row, pl.ds(oh + c * 16, 16)]
                        tw = t_rows[row, pl.ds(ot + c * 16, 16)]
                        rw = rel_cache[rrow, pl.ds(orr + c * 16, 16)]
                        h0, h1 = plsc.unpack(
                            plsc.bitcast(hw, jnp.bfloat16),
                            format=plsc.PackFormat.INTERLEAVED)
                        t0, t1 = plsc.unpack(
                            plsc.bitcast(tw, jnp.bfloat16),
                            format=plsc.PackFormat.INTERLEAVED)
                        r0, r1 = plsc.unpack(
                            plsc.bitcast(rw, jnp.bfloat16),
                            format=plsc.PackFormat.INTERLEAVED)
                        p = h0 * t0 * r0 + h1 * t1 * r1
                        acc = p if acc is None else acc + p
                        sq_acc = sq_acc + (h0 * h0 + h1 * h1 + t0 * t0
                                           + t1 * t1 + r0 * r0 + r1 * r1)
                    svec = jnp.where(lane == j, jnp.sum(acc), svec)
                dots_v[pl.ds(chunk * CHUNK + g * 16, 16)] = svec
                return sq_acc

            sq_acc = lax.fori_loop(0, CHUNK // 16, group_body, sq_acc)
        pltpu.sync_copy(dots_v, dots_hbm.at[pl.ds(wid * PER_W, PER_W)])
        sq_v[...] = sq_acc
        pltpu.sync_copy(sq_v, sq_hbm.at[wid])

    return sc_kernel(h2, offh, t2, offt, r2, offr, ent2, rel2)


_EQ = 12800
_RQ = 256


def _tc_transpose_quads(tab_t, n_rows, q):
    grid = (n_rows + 4 * q - 1) // (4 * q)

    def body(b0, b1, b2, b3, out_ref):
        words = []
        for ref in (b0, b1, b2, b3):
            u = lax.bitcast_convert_type(ref[...], jnp.uint32)
            r = (u + 0x7FFF + ((u >> 16) & 1)) >> 16
            words.append((r[32:64, :] << 16) | r[0:32, :])
        stacked = jnp.concatenate(words, axis=0)
        out_ref[...] = jnp.transpose(
            lax.bitcast_convert_type(stacked, jnp.int32))

    def qmap(k):
        def f(i):
            j = 4 * i + k
            return (0, jnp.where(j * q >= n_rows, 0, j))
        return f

    return pl.pallas_call(
        body,
        grid=(grid,),
        in_specs=[pl.BlockSpec((64, q), qmap(k)) for k in range(4)],
        out_specs=pl.BlockSpec((q, 128), lambda i: (i, 0)),
        out_shape=jax.ShapeDtypeStruct((grid * q, 128), jnp.int32),
    )(tab_t, tab_t, tab_t, tab_t)


def _finalize(dots, sq):
    rows = B2 // 128

    def body(dots_ref, sq_ref, out_ref):
        s = dots_ref[...]
        rowid = lax.broadcasted_iota(jnp.int32, (rows, 128), 0)
        x = jnp.where(rowid < rows // 2, -s, s)
        sp = jnp.maximum(x, 0.0) + jnp.log1p(jnp.exp(-jnp.abs(x)))
        mean_sp = jnp.sum(sp) / float(B2)
        regul = jnp.sum(sq_ref[...]) / float(B2 * D)
        out_ref[...] = jnp.reshape(mean_sp + LMBDA * regul, (1, 1))

    return pl.pallas_call(
        body,
        out_shape=jax.ShapeDtypeStruct((1, 1), jnp.float32),
    )(dots.reshape(rows, 128), sq)


def kernel(pos_h, pos_r, pos_t, neg_h, neg_r, neg_t, entity_emb, relation_emb):
    h_idx = jnp.concatenate([pos_h, neg_h])
    t_idx = jnp.concatenate([pos_t, neg_t])
    r_idx = jnp.concatenate([pos_r[:, 0], neg_r[:, 0]])

    def split(idx, q, halved):
        row = (idx // (4 * q)) * q + (idx % q)
        quarter = (idx // q) % 4
        if halved:
            row = 2 * row + quarter // 2
            off = (quarter % 2) * 32
        else:
            off = quarter * 32
        return (jnp.reshape(row, (B2 // IDX_W, IDX_W)),
                jnp.reshape(off, (B2 // IDX_W, IDX_W)))

    h2, offh = split(h_idx, _EQ, True)
    t2, offt = split(t_idx, _EQ, True)
    r2, offr = split(r_idx, _RQ, False)
    ent2 = _tc_transpose_quads(entity_emb.T, 1000000, _EQ).reshape(-1, D)
    rel2 = _tc_transpose_quads(relation_emb.T, 1000, _RQ)
    dots, sq = _sc_gather_score(h2, offh, t2, offt, r2, offr, ent2, rel2)
    return _finalize(dots, sq)[0, 0]
